# baseline (speedup 1.0000x reference)
import functools

import jax
import jax.numpy as jnp
from jax import lax
from jax.experimental import pallas as pl
from jax.experimental.pallas import tpu as pltpu
from jax.experimental.pallas import tpu_sc as plsc

_SPLITS = (1021, 1023, 1024, 1019, 1025, 1022, 1026, 1024)
_ALIGN = 16
_F = 2048
_PADDED = tuple((m + _ALIGN - 1) // _ALIGN * _ALIGN for m in _SPLITS)
_TOTAL_IN = sum(_SPLITS)
_TOTAL_OUT = sum(_PADDED)
_NW = 32
_WPE = _NW // len(_SPLITS)

_CH = 16


def _worker_params():
    src_off, dst_off = [], []
    s = d = 0
    for m, pm in zip(_SPLITS, _PADDED):
        src_off.append(s)
        dst_off.append(d)
        s += m
        d += pm
    n_l, src0_l, dst0_l, pad0_l, pend_l = [], [], [], [], []
    for w in range(_NW):
        e, q = divmod(w, _WPE)
        m, pm = _SPLITS[e], _PADDED[e]
        base = (pm // _WPE) // 16 * 16
        sizes = [base] * _WPE
        sizes[0] += pm - base * _WPE
        lo = dst_off[e] + sum(sizes[:q])
        shift = src_off[e] - dst_off[e]
        n_l.append(sizes[q] // _CH)
        src0_l.append(lo + shift)
        dst0_l.append(lo)
        has_pad = q == _WPE - 1 and pm != m
        pad0_l.append(dst_off[e] + m if has_pad else -1)
        pend_l.append(dst_off[e] + pm)
    return n_l, src0_l, dst0_l, pad0_l, pend_l


_N_L, _SRC0_L, _DST0_L, _PAD0_L, _PEND_L = _worker_params()


@functools.partial(
    pl.kernel,
    mesh=plsc.VectorSubcoreMesh(core_axis_name="c", subcore_axis_name="s"),
    out_type=jax.ShapeDtypeStruct((_TOTAL_OUT, _F), jnp.float32),
    scratch_types=[
        pltpu.VMEM((3, _CH, _F), jnp.float32),
        pltpu.SemaphoreType.DMA,
        pltpu.SemaphoreType.DMA,
        pltpu.SemaphoreType.DMA,
        pltpu.SemaphoreType.DMA,
        pltpu.SemaphoreType.DMA,
        pltpu.SemaphoreType.DMA,
    ],
)
def _pad_rows(inp_hbm, zeros_hbm, out_hbm, buf, g0, g1, g2, s0, s1, s2):
    wid = lax.axis_index("s") * 2 + lax.axis_index("c")
    gsem = (g0, g1, g2)
    ssem = (s0, s1, s2)
    iota = lax.iota(jnp.int32, 16)

    def _sel(vals):
        x = jnp.int32(vals[0])
        for w in range(1, _NW):
            x = jnp.where(wid == w, jnp.int32(vals[w]), x)
        return x

    n = _sel(_N_L)
    src0 = _sel(_SRC0_L)
    dst0 = _sel(_DST0_L)
    pad0 = _sel(_PAD0_L)
    pend = _sel(_PEND_L)

    def _g_start(i, b):
        pltpu.async_copy(inp_hbm.at[iota + (src0 + _CH * i)], buf.at[b],
                         gsem[b])

    def _g_wait(b):
        pltpu.make_async_copy(inp_hbm.at[pl.ds(0, _CH)], buf.at[b],
                              gsem[b]).wait()

    def _s_start(i, b):
        dst = pl.multiple_of(dst0 + _CH * i, 8)
        pltpu.async_copy(buf.at[b], out_hbm.at[pl.ds(dst, _CH)], ssem[b])

    def _s_wait(b):
        pltpu.make_async_copy(buf.at[b], out_hbm.at[pl.ds(0, _CH)],
                              ssem[b]).wait()

    def _step(j, carry):
        for b in range(3):
            i = 3 * j + b

            @pl.when((i >= 3) & (i <= n + 2))
            def _():
                _s_wait(b)

            @pl.when(i < n)
            def _():
                _g_start(i, b)

            @pl.when((i >= 1) & (i <= n))
            def _():
                bp = (b + 2) % 3
                _g_wait(bp)
                _s_start(i - 1, bp)

        return carry

    lax.fori_loop(0, (n + 6) // 3, _step, 0)

    @pl.when(pad0 >= 0)
    def _():
        pltpu.async_copy(zeros_hbm, buf.at[0], gsem[0]).wait()
        zidx = jnp.minimum(iota + pad0, pend - 1)
        pltpu.async_copy(buf.at[0], out_hbm.at[zidx], ssem[0]).wait()


def kernel(inp, m_splits):
    zeros = jnp.zeros((_CH, _F), dtype=inp.dtype)
    out = _pad_rows(inp, zeros)
    deltas = jnp.array([pm - m for m, pm in zip(_SPLITS, _PADDED)],
                       dtype=jnp.int64)
    return out, jnp.asarray(m_splits, dtype=jnp.int64) + deltas


# speedup vs baseline: 1.0074x; 1.0074x over previous
import functools

import jax
import jax.numpy as jnp
from jax import lax
from jax.experimental import pallas as pl
from jax.experimental.pallas import tpu as pltpu
from jax.experimental.pallas import tpu_sc as plsc

_SPLITS = (1021, 1023, 1024, 1019, 1025, 1022, 1026, 1024)
_ALIGN = 16
_F = 2048
_PADDED = tuple((m + _ALIGN - 1) // _ALIGN * _ALIGN for m in _SPLITS)
_TOTAL_IN = sum(_SPLITS)
_TOTAL_OUT = sum(_PADDED)
_NW = 32
_WPE = _NW // len(_SPLITS)

_CH = 16


def _worker_params():
    src_off, dst_off = [], []
    s = d = 0
    for m, pm in zip(_SPLITS, _PADDED):
        src_off.append(s)
        dst_off.append(d)
        s += m
        d += pm
    n_l, src0_l, dst0_l, pad0_l, pend_l = [], [], [], [], []
    for w in range(_NW):
        e, q = divmod(w, _WPE)
        m, pm = _SPLITS[e], _PADDED[e]
        base = (pm // _WPE) // 16 * 16
        sizes = [base] * _WPE
        sizes[0] += pm - base * _WPE
        lo = dst_off[e] + sum(sizes[:q])
        shift = src_off[e] - dst_off[e]
        n_l.append(sizes[q] // _CH)
        src0_l.append(lo + shift)
        dst0_l.append(lo)
        has_pad = q == _WPE - 1 and pm != m
        pad0_l.append(dst_off[e] + m if has_pad else -1)
        pend_l.append(dst_off[e] + pm)
    return n_l, src0_l, dst0_l, pad0_l, pend_l


_N_L, _SRC0_L, _DST0_L, _PAD0_L, _PEND_L = _worker_params()


@functools.partial(
    pl.kernel,
    mesh=plsc.VectorSubcoreMesh(core_axis_name="c", subcore_axis_name="s"),
    out_type=jax.ShapeDtypeStruct((_TOTAL_OUT, _F), jnp.float32),
    scratch_types=[
        pltpu.VMEM((3, _CH, _F), jnp.float32),
        pltpu.SemaphoreType.DMA,
        pltpu.SemaphoreType.DMA,
        pltpu.SemaphoreType.DMA,
        pltpu.SemaphoreType.DMA,
        pltpu.SemaphoreType.DMA,
        pltpu.SemaphoreType.DMA,
    ],
)
def _pad_rows(inp_hbm, out_hbm, buf, g0, g1, g2, s0, s1, s2):
    wid = lax.axis_index("s") * 2 + lax.axis_index("c")
    gsem = (g0, g1, g2)
    ssem = (s0, s1, s2)
    iota = lax.iota(jnp.int32, 16)
    zv = jnp.zeros((16,), jnp.float32)

    def _sel(vals):
        x = jnp.int32(vals[0])
        for w in range(1, _NW):
            x = jnp.where(wid == w, jnp.int32(vals[w]), x)
        return x

    n = _sel(_N_L)
    src0 = _sel(_SRC0_L)
    dst0 = _sel(_DST0_L)
    pad0 = _sel(_PAD0_L)

    def _g_start(i, b):
        pltpu.async_copy(inp_hbm.at[iota + (src0 + _CH * i)], buf.at[b],
                         gsem[b])

    def _g_wait(b):
        pltpu.make_async_copy(inp_hbm.at[pl.ds(0, _CH)], buf.at[b],
                              gsem[b]).wait()

    def _s_start(i, b):
        dst = pl.multiple_of(dst0 + _CH * i, 8)
        pltpu.async_copy(buf.at[b], out_hbm.at[pl.ds(dst, _CH)], ssem[b])

    def _s_wait(b):
        pltpu.make_async_copy(buf.at[b], out_hbm.at[pl.ds(0, _CH)],
                              ssem[b]).wait()

    def _step(j, carry):
        for b in range(3):
            i = 3 * j + b

            @pl.when((i >= 3) & (i <= n + 2))
            def _():
                _s_wait(b)

            @pl.when(i < n)
            def _():
                _g_start(i, b)

            @pl.when((i >= 1) & (i <= n))
            def _():
                bp = (b + 2) % 3
                _g_wait(bp)

                @pl.when((pad0 >= 0) & (i == n))
                def _():
                    rel0 = pad0 - (dst0 + _CH * (n - 1))

                    def _zrow(r, cc):
                        for c0 in range(_F // 16):
                            buf[bp, r, pl.ds(c0 * 16, 16)] = zv
                        return cc

                    lax.fori_loop(rel0, _CH, _zrow, 0)

                _s_start(i - 1, bp)

        return carry

    lax.fori_loop(0, (n + 6) // 3, _step, 0)


def kernel(inp, m_splits):
    out = _pad_rows(inp)
    deltas = jnp.array([pm - m for m, pm in zip(_SPLITS, _PADDED)],
                       dtype=jnp.int64)
    return out, jnp.asarray(m_splits, dtype=jnp.int64) + deltas


# speedup vs baseline: 1.0114x; 1.0040x over previous
import functools

import jax
import jax.numpy as jnp
from jax import lax
from jax.experimental import pallas as pl
from jax.experimental.pallas import tpu as pltpu
from jax.experimental.pallas import tpu_sc as plsc

_SPLITS = (1021, 1023, 1024, 1019, 1025, 1022, 1026, 1024)
_ALIGN = 16
_F = 2048
_PADDED = tuple((m + _ALIGN - 1) // _ALIGN * _ALIGN for m in _SPLITS)
_TOTAL_IN = sum(_SPLITS)
_TOTAL_OUT = sum(_PADDED)
_NW = 32
_WPE = _NW // len(_SPLITS)

_CH = 16


def _worker_params():
    src_off, dst_off = [], []
    s = d = 0
    for m, pm in zip(_SPLITS, _PADDED):
        src_off.append(s)
        dst_off.append(d)
        s += m
        d += pm
    n_l, src0_l, dst0_l, pad0_l, pend_l = [], [], [], [], []
    for w in range(_NW):
        e, q = divmod(w, _WPE)
        m, pm = _SPLITS[e], _PADDED[e]
        base = (pm // _WPE) // 16 * 16
        sizes = [base] * _WPE
        sizes[0] += pm - base * _WPE
        lo = dst_off[e] + sum(sizes[:q])
        shift = src_off[e] - dst_off[e]
        n_l.append(sizes[q] // _CH)
        src0_l.append(lo + shift)
        dst0_l.append(lo)
        has_pad = q == _WPE - 1 and pm != m
        pad0_l.append(dst_off[e] + m if has_pad else -1)
        pend_l.append(dst_off[e] + pm)
    perm = list(range(_NW))
    perm[24], perm[25] = 25, 24
    perm[18], perm[19] = 19, 18
    return tuple([l[j] for j in perm] for l in
                 (n_l, src0_l, dst0_l, pad0_l, pend_l))


_N_L, _SRC0_L, _DST0_L, _PAD0_L, _PEND_L = _worker_params()


@functools.partial(
    pl.kernel,
    mesh=plsc.VectorSubcoreMesh(core_axis_name="c", subcore_axis_name="s"),
    out_type=jax.ShapeDtypeStruct((_TOTAL_OUT, _F), jnp.float32),
    scratch_types=[
        pltpu.VMEM((3, _CH, _F), jnp.float32),
        pltpu.SemaphoreType.DMA,
        pltpu.SemaphoreType.DMA,
        pltpu.SemaphoreType.DMA,
        pltpu.SemaphoreType.DMA,
        pltpu.SemaphoreType.DMA,
        pltpu.SemaphoreType.DMA,
    ],
)
def _pad_rows(inp_hbm, out_hbm, buf, g0, g1, g2, s0, s1, s2):
    wid = lax.axis_index("s") * 2 + lax.axis_index("c")
    gsem = (g0, g1, g2)
    ssem = (s0, s1, s2)
    iota = lax.iota(jnp.int32, 16)
    zv = jnp.zeros((16,), jnp.float32)

    def _sel(vals):
        x = jnp.int32(vals[0])
        for w in range(1, _NW):
            x = jnp.where(wid == w, jnp.int32(vals[w]), x)
        return x

    n = _sel(_N_L)
    src0 = _sel(_SRC0_L)
    dst0 = _sel(_DST0_L)
    pad0 = _sel(_PAD0_L)

    def _g_start(i, b):
        pltpu.async_copy(inp_hbm.at[iota + (src0 + _CH * i)], buf.at[b],
                         gsem[b])

    def _g_wait(b):
        pltpu.make_async_copy(inp_hbm.at[pl.ds(0, _CH)], buf.at[b],
                              gsem[b]).wait()

    def _s_start(i, b):
        dst = pl.multiple_of(dst0 + _CH * i, 8)
        pltpu.async_copy(buf.at[b], out_hbm.at[pl.ds(dst, _CH)], ssem[b])

    def _s_wait(b):
        pltpu.make_async_copy(buf.at[b], out_hbm.at[pl.ds(0, _CH)],
                              ssem[b]).wait()

    def _step(j, carry):
        for b in range(3):
            i = 3 * j + b

            @pl.when((i >= 3) & (i <= n + 2))
            def _():
                _s_wait(b)

            @pl.when(i < n)
            def _():
                _g_start(i, b)

            @pl.when((i >= 1) & (i <= n))
            def _():
                bp = (b + 2) % 3
                _g_wait(bp)

                @pl.when((pad0 >= 0) & (i == n))
                def _():
                    rel0 = pad0 - (dst0 + _CH * (n - 1))

                    def _zrow(r, cc):
                        for c0 in range(_F // 16):
                            buf[bp, r, pl.ds(c0 * 16, 16)] = zv
                        return cc

                    lax.fori_loop(rel0, _CH, _zrow, 0)

                _s_start(i - 1, bp)

        return carry

    lax.fori_loop(0, (n + 6) // 3, _step, 0)


def kernel(inp, m_splits):
    out = _pad_rows(inp)
    deltas = jnp.array([pm - m for m, pm in zip(_SPLITS, _PADDED)],
                       dtype=jnp.int64)
    return out, jnp.asarray(m_splits, dtype=jnp.int64) + deltas
